# baseline (speedup 1.0000x reference)
import functools

import jax
import jax.numpy as jnp
from jax import lax
from jax.experimental import pallas as pl
from jax.experimental.pallas import tpu as pltpu
from jax.experimental.pallas import tpu_sc as plsc

_B = 4096
_N = 20
_D = 64
_NC = 2
_NS = 16
_NW = _NC * _NS
_BPW = _B // _NW
_CHUNK = 32
_NCHUNK = _BPW // _CHUNK
_IDX = _CHUNK * _N
_G = _IDX // 128
_IROWS = _B * _N // 128
_NEG = -1e30


def _make_sc_kernel():
    mesh = plsc.VectorSubcoreMesh(core_axis_name="c", subcore_axis_name="s")

    @functools.partial(
        pl.kernel,
        out_type=jax.ShapeDtypeStruct((_B, 32), jnp.float32),
        mesh=mesh,
        scratch_types=[
            pltpu.VMEM((_IDX,), jnp.int32),
            pltpu.VMEM((_IDX, _D), jnp.float32),
            pltpu.VMEM((_BPW, _D), jnp.float32),
            pltpu.VMEM((_CHUNK, 32), jnp.float32),
            pltpu.VMEM((32, 16), jnp.float32),
            pltpu.SemaphoreType.DMA,
        ],
        compiler_params=pltpu.CompilerParams(
            use_tc_tiling_on_sc=False,
            needs_layout_passes=False,
        ),
    )
    def sc_kernel(ctx_hbm, ids_hbm, table_hbm, out_hbm,
                  idx_v, rows_v, ctx_v, out_v, sc_v, sem):
        wid = lax.axis_index("s") * _NC + lax.axis_index("c")
        neg = jnp.full((16,), _NEG, jnp.float32)
        for n in range(_N, 32):
            sc_v[n] = neg
        pltpu.sync_copy(ctx_hbm.at[pl.ds(wid * _BPW, _BPW)], ctx_v)

        for g in range(_NCHUNK):
            b_base = wid * _BPW + g * _CHUNK
            i_base = (wid * _BPW + g * _CHUNK) * _N
            pltpu.sync_copy(ids_hbm.at[pl.ds(i_base, _IDX)], idx_v)
            copies = [
                pltpu.async_copy(
                    table_hbm.at[idx_v.at[pl.ds(j * 128, 128)]],
                    rows_v.at[pl.ds(j * 128, 128)],
                    sem,
                )
                for j in range(_G)
            ]
            for cpy in copies:
                cpy.wait()

            def body(b, carry, g=g):
                bl = g * _CHUNK + b
                c = [ctx_v[bl, pl.ds(16 * k, 16)] for k in range(4)]
                for n in range(_N):
                    r = b * _N + n
                    acc = rows_v[r, pl.ds(0, 16)] * c[0]
                    for k in range(1, 4):
                        acc = acc + rows_v[r, pl.ds(16 * k, 16)] * c[k]
                    sc_v[n] = lax.broadcast_in_dim(jnp.sum(acc), (16,), ())
                row_ids = lax.iota(jnp.int32, 16)
                col_ids = jnp.zeros((16,), jnp.int32)
                v0 = plsc.load_gather(sc_v, [row_ids, col_ids])
                v1 = plsc.load_gather(sc_v, [row_ids + 16, col_ids])
                m = jnp.maximum(jnp.max(v0), jnp.max(v1))
                e0 = jnp.exp(v0 - m)
                e1 = jnp.exp(v1 - m)
                tot = lax.broadcast_in_dim(jnp.sum(e0) + jnp.sum(e1), (16,), ())
                out_v[b, pl.ds(0, 16)] = e0 / tot
                out_v[b, pl.ds(16, 16)] = e1 / tot
                return carry

            lax.fori_loop(0, _CHUNK, body, 0)
            pltpu.sync_copy(out_v, out_hbm.at[pl.ds(b_base, _CHUNK)])

    return sc_kernel


_SC_KERNEL = _make_sc_kernel()


def kernel(context_encoded, entity_ids, entity_embeddings):
    ids_flat = entity_ids.reshape(_B * _N)
    out = _SC_KERNEL(context_encoded, ids_flat, entity_embeddings)
    return out[:, :_N]
